# Initial kernel scaffold; baseline (speedup 1.0000x reference)
import jax
import jax.numpy as jnp
from jax.experimental import pallas as pl


def kernel(hidden_states, positions, kv_cache, slot_mapping, seq_lens, rms1_w, rms2_w, q_w, kv_w, o_w, router_w, w1, w1_up, w2, sg_w, su_w, sd_w):
    raise NotImplementedError("write your pallas kernel here")


# speedup vs baseline: 1.6400x; 1.6400x over previous
import jax
import jax.numpy as jnp
from jax import lax
from jax.experimental import pallas as pl
from jax.experimental.pallas import tpu as pltpu

B = 8
H = 16
NOPE = 128
ROPE = 64
VD = 128
HID = 2048
KV = 2048
E = 8
TOPK = 2
INTER = 1408
EPS = 1e-5

F32 = jnp.float32
_DOT = dict(preferred_element_type=jnp.float32)

QBLK = 384
KVUSED = H * (NOPE + VD)
KVBLK = 512
NPROJ = 8
OBLK = 256
NI = 11
IBLK = INTER // NI


def _rms(x, w):
    var = jnp.mean(x * x, axis=-1, keepdims=True)
    return x * lax.rsqrt(var + EPS) * w


def _proj_qkv_body(hs_ref, w_ref, qw_ref, kvw_ref, q_ref, kv_ref):
    xn = _rms(hs_ref[:, 0, :], w_ref[...])
    q_ref[...] = lax.dot_general(xn, qw_ref[...], (((1,), (1,)), ((), ())), **_DOT)
    kv_ref[...] = lax.dot_general(xn, kvw_ref[...], (((1,), (1,)), ((), ())), **_DOT)


def _proj_qkv(hs, rms1_w, q_w, kv_w):
    return pl.pallas_call(
        _proj_qkv_body,
        grid=(NPROJ,),
        in_specs=[
            pl.BlockSpec((B, 1, HID), lambda g: (0, 0, 0)),
            pl.BlockSpec((1, HID), lambda g: (0, 0)),
            pl.BlockSpec((QBLK, HID), lambda g: (g, 0)),
            pl.BlockSpec((KVBLK, HID), lambda g: (g, 0)),
        ],
        out_specs=[
            pl.BlockSpec((B, QBLK), lambda g: (0, g)),
            pl.BlockSpec((B, KVBLK), lambda g: (0, g)),
        ],
        out_shape=[
            jax.ShapeDtypeStruct((B, H * (NOPE + ROPE)), F32),
            jax.ShapeDtypeStruct((B, KVUSED), F32),
        ],
    )(hs, rms1_w, q_w, kv_w)


SBLK = 512
NS = KV // SBLK


def _attn_body(slot_ref, seq_ref, q_ref, k_ref, v_ref, cache_ref, out_ref,
               m_s, l_s, acc_ref):
    b = pl.program_id(0)
    s = pl.program_id(1)
    slot = slot_ref[b]
    sl = seq_ref[b]
    iota = lax.broadcasted_iota(jnp.int32, (SBLK, 1), 0) + s * SBLK
    at_slot = iota == slot
    valid = iota < sl

    @pl.when(s == 0)
    def _():
        acc_ref[...] = jnp.zeros((H, VD), F32)
        for h in range(H):
            m_s[h] = -1e30
            l_s[h] = 0.0

    rows = []
    for h in range(H):
        kc = cache_ref[0, :, h, :NOPE]
        vc = cache_ref[0, :, h, NOPE:]
        q2 = q_ref[0, h:h + 1, :]
        sc = lax.dot_general(kc, q2, (((1,), (1,)), ((), ())), **_DOT)
        qk_new = lax.dot_general(k_ref[0, h:h + 1, :], q2,
                                 (((1,), (1,)), ((), ())), **_DOT)
        sc = jnp.where(at_slot, qk_new, sc) * (NOPE ** -0.5)
        sc = jnp.where(valid, sc, -1e9)
        m_old = m_s[h]
        l_old = l_s[h]
        m_new = jnp.maximum(m_old, jnp.max(sc))
        corr = jnp.exp(m_old - m_new)
        p = jnp.where(valid, jnp.exp(sc - m_new), 0.0)
        l_new = l_old * corr + jnp.sum(p)
        vc2 = jnp.where(at_slot, v_ref[0, h:h + 1, :], vc)
        ctx = lax.dot_general(p, vc2, (((0,), (0,)), ((), ())), **_DOT)
        acc_new = acc_ref[h:h + 1, :] * corr + ctx
        acc_ref[h:h + 1, :] = acc_new
        m_s[h] = m_new
        l_s[h] = l_new
        rows.append(acc_new / l_new)
    out_ref[0] = jnp.concatenate(rows, axis=0)


def _attn(q3, k3, v3, kv_cache, slot_mapping, seq_lens):
    return pl.pallas_call(
        _attn_body,
        grid=(B, NS),
        in_specs=[
            pl.BlockSpec(memory_space=pltpu.SMEM),
            pl.BlockSpec(memory_space=pltpu.SMEM),
            pl.BlockSpec((1, H, NOPE), lambda b, s: (b, 0, 0)),
            pl.BlockSpec((1, H, NOPE), lambda b, s: (b, 0, 0)),
            pl.BlockSpec((1, H, VD), lambda b, s: (b, 0, 0)),
            pl.BlockSpec((1, SBLK, H, NOPE + VD), lambda b, s: (b, s, 0, 0)),
        ],
        out_specs=pl.BlockSpec((1, H, VD), lambda b, s: (b, 0, 0)),
        out_shape=jax.ShapeDtypeStruct((B, H, VD), F32),
        scratch_shapes=[
            pltpu.SMEM((H,), F32),
            pltpu.SMEM((H,), F32),
            pltpu.VMEM((H, VD), F32),
        ],
    )(slot_mapping, seq_lens, q3, k3, v3, kv_cache)


def _oproj_body(ctx_ref, ow_ref, out_ref):
    out_ref[...] = lax.dot_general(
        ctx_ref[...], ow_ref[...], (((1,), (1,)), ((), ())), **_DOT)


def _oproj(ctxflat, o_w):
    return pl.pallas_call(
        _oproj_body,
        grid=(HID // OBLK,),
        in_specs=[
            pl.BlockSpec((B, H * VD), lambda g: (0, 0)),
            pl.BlockSpec((OBLK, H * VD), lambda g: (g, 0)),
        ],
        out_specs=pl.BlockSpec((B, OBLK), lambda g: (0, g)),
        out_shape=jax.ShapeDtypeStruct((B, HID), F32),
    )(ctxflat, o_w)


def _router_body(ao_ref, hs_ref, w_ref, rw_ref, resid_ref, h2_ref, coef_ref):
    resid = hs_ref[:, 0, :] + ao_ref[...]
    resid_ref[...] = resid
    h2 = _rms(resid, w_ref[...])
    h2_ref[...] = h2
    lt = lax.dot_general(rw_ref[...], h2, (((1,), (1,)), ((), ())), **_DOT)
    ei = lax.broadcasted_iota(jnp.int32, (E, B), 0)
    m1 = jnp.max(lt, axis=0, keepdims=True)
    i1 = jnp.min(jnp.where(lt == m1, ei, E + 1), axis=0, keepdims=True)
    oh1 = ei == i1
    lt2 = jnp.where(oh1, -1e30, lt)
    m2 = jnp.max(lt2, axis=0, keepdims=True)
    i2 = jnp.min(jnp.where(lt2 == m2, ei, E + 1), axis=0, keepdims=True)
    oh2 = ei == i2
    z = jnp.exp(m2 - m1)
    wa = 1.0 / (1.0 + z)
    wb = z / (1.0 + z)
    coef = jnp.where(oh1, wa, 0.0) + jnp.where(oh2, wb, 0.0)
    coef_ref[...] = jnp.concatenate(
        [coef, jnp.ones((1, B), F32), jnp.zeros((2 * E - 1 - E, B), F32)],
        axis=0)


def _router(attn_out, hs, rms2_w, router_w):
    return pl.pallas_call(
        _router_body,
        grid=(1,),
        in_specs=[
            pl.BlockSpec((B, HID), lambda g: (0, 0)),
            pl.BlockSpec((B, 1, HID), lambda g: (0, 0, 0)),
            pl.BlockSpec((1, HID), lambda g: (0, 0)),
            pl.BlockSpec((E, HID), lambda g: (0, 0)),
        ],
        out_specs=[
            pl.BlockSpec((B, HID), lambda g: (0, 0)),
            pl.BlockSpec((B, HID), lambda g: (0, 0)),
            pl.BlockSpec((2 * E, B), lambda g: (0, 0)),
        ],
        out_shape=[
            jax.ShapeDtypeStruct((B, HID), F32),
            jax.ShapeDtypeStruct((B, HID), F32),
            jax.ShapeDtypeStruct((2 * E, B), F32),
        ],
    )(attn_out, hs, rms2_w, router_w)


def _silu(x):
    return x * (1.0 / (1.0 + jnp.exp(-x)))


def _moe_body(coef_ref, h2_ref, resid_ref, w1_ref, w1u_ref, w2_ref,
              sg_ref, su_ref, sd_ref, out_ref):
    i = pl.program_id(0)
    e = pl.program_id(1)
    h2 = h2_ref[...]

    @pl.when(jnp.logical_and(i == 0, e == 0))
    def _():
        out_ref[...] = resid_ref[...]

    @pl.when(e < E)
    def _():
        g = lax.dot_general(h2, w1_ref[0], (((1,), (1,)), ((), ())), **_DOT)
        u = lax.dot_general(h2, w1u_ref[0], (((1,), (1,)), ((), ())), **_DOT)
        eo = lax.dot_general(_silu(g) * u, w2_ref[0],
                             (((1,), (1,)), ((), ())), **_DOT)
        bi = lax.broadcasted_iota(jnp.int32, (B, 1), 0)
        c = jnp.zeros((B, 1), F32)
        for b in range(B):
            c = jnp.where(bi == b, coef_ref[e, b], c)
        out_ref[...] += eo * c

    @pl.when(e == E)
    def _():
        g = lax.dot_general(h2, sg_ref[...], (((1,), (1,)), ((), ())), **_DOT)
        u = lax.dot_general(h2, su_ref[...], (((1,), (1,)), ((), ())), **_DOT)
        eo = lax.dot_general(_silu(g) * u, sd_ref[...],
                             (((1,), (1,)), ((), ())), **_DOT)
        out_ref[...] += eo


def _moe(coefT, h2, resid, w1, w1_up, w2, sg_w, su_w, sd_w):
    clamp = lambda e: jnp.minimum(e, E - 1)
    return pl.pallas_call(
        _moe_body,
        grid=(NI, E + 1),
        in_specs=[
            pl.BlockSpec(memory_space=pltpu.SMEM),
            pl.BlockSpec((B, HID), lambda i, e: (0, 0)),
            pl.BlockSpec((B, HID), lambda i, e: (0, 0)),
            pl.BlockSpec((1, IBLK, HID), lambda i, e: (clamp(e), i, 0)),
            pl.BlockSpec((1, IBLK, HID), lambda i, e: (clamp(e), i, 0)),
            pl.BlockSpec((1, HID, IBLK), lambda i, e: (clamp(e), 0, i)),
            pl.BlockSpec((IBLK, HID), lambda i, e: (i, 0)),
            pl.BlockSpec((IBLK, HID), lambda i, e: (i, 0)),
            pl.BlockSpec((HID, IBLK), lambda i, e: (0, i)),
        ],
        out_specs=pl.BlockSpec((B, HID), lambda i, e: (0, 0)),
        out_shape=jax.ShapeDtypeStruct((B, HID), F32),
    )(coefT, h2, resid, w1, w1_up, w2, sg_w, su_w, sd_w)


def kernel(hidden_states, positions, kv_cache, slot_mapping, seq_lens,
           rms1_w, rms2_w, q_w, kv_w, o_w, router_w, w1, w1_up, w2,
           sg_w, su_w, sd_w):
    q_all, kv_all = _proj_qkv(hidden_states, rms1_w.reshape(1, HID), q_w, kv_w)
    q3 = q_all.reshape(B, H, NOPE + ROPE)[:, :, :NOPE]
    k3 = kv_all[:, :H * NOPE].reshape(B, H, NOPE)
    v3 = kv_all[:, H * NOPE:H * (NOPE + VD)].reshape(B, H, VD)
    ctx = _attn(q3, k3, v3, kv_cache, slot_mapping, seq_lens)
    attn_out = _oproj(ctx.reshape(B, H * VD), o_w)
    resid, h2, coefT = _router(attn_out, hidden_states,
                               rms2_w.reshape(1, HID), router_w)
    out = _moe(coefT, h2, resid, w1, w1_up, w2, sg_w, su_w, sd_w)
    return out[:, None, :]
